# Initial kernel scaffold; baseline (speedup 1.0000x reference)
import jax
import jax.numpy as jnp
from jax.experimental import pallas as pl


def kernel(x, W_E):
    raise NotImplementedError("write your pallas kernel here")


# speedup vs baseline: 1.5406x; 1.5406x over previous
import functools

import jax
import jax.numpy as jnp
from jax import lax
from jax.experimental import pallas as pl
from jax.experimental.pallas import tpu as pltpu
from jax.experimental.pallas import tpu_sc as plsc


def _make_gather(V, D, B):
    info = plsc.get_sparse_core_info()
    NC, NS = info.num_cores, info.num_subcores
    NW = NC * NS
    assert B % NW == 0
    b_per_w = B // NW
    CH = 32
    NBUF = 2
    nchunk = b_per_w // CH
    assert nchunk >= NBUF

    mesh = plsc.VectorSubcoreMesh(core_axis_name="c", subcore_axis_name="s")

    @functools.partial(
        pl.kernel,
        mesh=mesh,
        out_type=jax.ShapeDtypeStruct((B, D), jnp.float32),
        scratch_types=[
            pltpu.VMEM((b_per_w,), jnp.int32),
            pltpu.VMEM((NBUF, CH, D), jnp.float32),
            pltpu.SemaphoreType.DMA,
            pltpu.SemaphoreType.DMA,
            pltpu.SemaphoreType.DMA,
            pltpu.SemaphoreType.DMA,
        ],
    )
    def k(table_hbm, idx_hbm, out_hbm, idx_v, rows_v, g0, g1, w0, w1):
        wid = lax.axis_index("s") * NC + lax.axis_index("c")
        base = wid * b_per_w
        pltpu.sync_copy(idx_hbm.at[pl.ds(base, b_per_w)], idx_v)

        gsem = [g0, g1]
        wsem = [w0, w1]

        def gather_start(g):
            return pltpu.async_copy(
                table_hbm.at[idx_v.at[pl.ds(g * CH, CH)]],
                rows_v.at[g % NBUF],
                gsem[g % NBUF],
            )

        def write_start(g):
            return pltpu.async_copy(
                rows_v.at[g % NBUF],
                out_hbm.at[pl.ds(base + g * CH, CH)],
                wsem[g % NBUF],
            )

        g_h = {0: gather_start(0)}
        w_h = {}
        for g in range(nchunk):
            if g + 1 < nchunk:
                if g + 1 >= NBUF:
                    w_h[g + 1 - NBUF].wait()
                g_h[g + 1] = gather_start(g + 1)
            g_h[g].wait()
            w_h[g] = write_start(g)
        for g in range(max(0, nchunk - NBUF), nchunk):
            w_h[g].wait()

    return k


def kernel(x, W_E):
    s, V, D = W_E.shape
    b, c = x.shape
    B = b * c
    table = W_E.reshape(V, D)
    idx = x.reshape(B).astype(jnp.int32)
    out = _make_gather(V, D, B)(table, idx)
    return out.reshape(b, s, c, D)


# speedup vs baseline: 1.5832x; 1.0277x over previous
import functools

import jax
import jax.numpy as jnp
from jax import lax
from jax.experimental import pallas as pl
from jax.experimental.pallas import tpu as pltpu
from jax.experimental.pallas import tpu_sc as plsc


def _make_gather(V, D, B):
    info = plsc.get_sparse_core_info()
    NC, NS = info.num_cores, info.num_subcores
    NW = NC * NS
    assert B % NW == 0
    b_per_w = B // NW
    CH = 32
    NBUF = 3
    nchunk = b_per_w // CH
    assert nchunk >= NBUF

    mesh = plsc.VectorSubcoreMesh(core_axis_name="c", subcore_axis_name="s")

    @functools.partial(
        pl.kernel,
        mesh=mesh,
        out_type=jax.ShapeDtypeStruct((B, D), jnp.float32),
        scratch_types=[
            pltpu.VMEM((b_per_w,), jnp.int32),
            pltpu.VMEM((NBUF, CH, D), jnp.float32),
        ] + [pltpu.SemaphoreType.DMA] * (2 * NBUF),
    )
    def k(table_hbm, idx_hbm, out_hbm, idx_v, rows_v, *sems):
        wid = lax.axis_index("s") * NC + lax.axis_index("c")
        base = wid * b_per_w
        pltpu.sync_copy(idx_hbm.at[pl.ds(base, b_per_w)], idx_v)

        gsem = sems[:NBUF]
        wsem = sems[NBUF:]

        def gather_start(g):
            return pltpu.async_copy(
                table_hbm.at[idx_v.at[pl.ds(g * CH, CH)]],
                rows_v.at[g % NBUF],
                gsem[g % NBUF],
            )

        def write_start(g):
            return pltpu.async_copy(
                rows_v.at[g % NBUF],
                out_hbm.at[pl.ds(base + g * CH, CH)],
                wsem[g % NBUF],
            )

        g_h = {}
        w_h = {}
        for g in range(min(NBUF, nchunk)):
            g_h[g] = gather_start(g)
        for g in range(nchunk):
            nxt = g + NBUF - 1
            if NBUF <= nxt < nchunk:
                w_h[nxt - NBUF].wait()
                g_h[nxt] = gather_start(nxt)
            g_h[g].wait()
            w_h[g] = write_start(g)
        for g in range(max(0, nchunk - NBUF), nchunk):
            w_h[g].wait()

    return k


def kernel(x, W_E):
    s, V, D = W_E.shape
    b, c = x.shape
    B = b * c
    table = W_E.reshape(V, D)
    idx = x.reshape(B).astype(jnp.int32)
    out = _make_gather(V, D, B)(table, idx)
    return out.reshape(b, s, c, D)


# speedup vs baseline: 1.5899x; 1.0042x over previous
import functools

import jax
import jax.numpy as jnp
from jax import lax
from jax.experimental import pallas as pl
from jax.experimental.pallas import tpu as pltpu
from jax.experimental.pallas import tpu_sc as plsc


def _make_gather(V, D, B):
    info = plsc.get_sparse_core_info()
    NC, NS = info.num_cores, info.num_subcores
    NW = NC * NS
    assert B % NW == 0
    b_per_w = B // NW
    CH = 16
    NBUF = 6
    nchunk = b_per_w // CH
    assert nchunk >= NBUF

    mesh = plsc.VectorSubcoreMesh(core_axis_name="c", subcore_axis_name="s")

    @functools.partial(
        pl.kernel,
        mesh=mesh,
        out_type=jax.ShapeDtypeStruct((B, D), jnp.float32),
        scratch_types=[
            pltpu.VMEM((b_per_w,), jnp.int32),
            pltpu.VMEM((NBUF, CH, D), jnp.float32),
        ] + [pltpu.SemaphoreType.DMA] * (2 * NBUF),
    )
    def k(table_hbm, idx_hbm, out_hbm, idx_v, rows_v, *sems):
        wid = lax.axis_index("s") * NC + lax.axis_index("c")
        base = wid * b_per_w
        pltpu.sync_copy(idx_hbm.at[pl.ds(base, b_per_w)], idx_v)

        gsem = sems[:NBUF]
        wsem = sems[NBUF:]

        def gather_start(g):
            return pltpu.async_copy(
                table_hbm.at[idx_v.at[pl.ds(g * CH, CH)]],
                rows_v.at[g % NBUF],
                gsem[g % NBUF],
            )

        def write_start(g):
            return pltpu.async_copy(
                rows_v.at[g % NBUF],
                out_hbm.at[pl.ds(base + g * CH, CH)],
                wsem[g % NBUF],
            )

        g_h = {}
        w_h = {}
        for g in range(min(NBUF, nchunk)):
            g_h[g] = gather_start(g)
        for g in range(nchunk):
            nxt = g + NBUF - 1
            if NBUF <= nxt < nchunk:
                w_h[nxt - NBUF].wait()
                g_h[nxt] = gather_start(nxt)
            g_h[g].wait()
            w_h[g] = write_start(g)
        for g in range(max(0, nchunk - NBUF), nchunk):
            w_h[g].wait()

    return k


def kernel(x, W_E):
    s, V, D = W_E.shape
    b, c = x.shape
    B = b * c
    table = W_E.reshape(V, D)
    idx = x.reshape(B).astype(jnp.int32)
    out = _make_gather(V, D, B)(table, idx)
    return out.reshape(b, s, c, D)


# speedup vs baseline: 1.6031x; 1.0083x over previous
import functools

import jax
import jax.numpy as jnp
from jax import lax
from jax.experimental import pallas as pl
from jax.experimental.pallas import tpu as pltpu
from jax.experimental.pallas import tpu_sc as plsc


def _make_gather(V, D, B):
    info = plsc.get_sparse_core_info()
    NC, NS = info.num_cores, info.num_subcores
    NW = NC * NS
    assert B % NW == 0
    b_per_w = B // NW
    CH = 16
    NBUF = 6
    nchunk = b_per_w // CH
    assert nchunk >= NBUF

    mesh = plsc.VectorSubcoreMesh(core_axis_name="c", subcore_axis_name="s")

    @functools.partial(
        pl.kernel,
        mesh=mesh,
        out_type=jax.ShapeDtypeStruct((B, D), jnp.float32),
        scratch_types=[
            pltpu.VMEM((b_per_w,), jnp.int32),
            pltpu.VMEM((NBUF, CH, D), jnp.float32),
        ] + [pltpu.SemaphoreType.DMA] * (2 * NBUF),
    )
    def k(table_hbm, idx_hbm, out_hbm, idx_v, rows_v, *sems):
        wid = lax.axis_index("s") * NC + lax.axis_index("c")
        base = wid * b_per_w
        per_row = idx_hbm.shape[1] // b_per_w
        pltpu.sync_copy(
            idx_hbm.at[wid // per_row, pl.ds((wid % per_row) * b_per_w, b_per_w)],
            idx_v,
        )

        gsem = sems[:NBUF]
        wsem = sems[NBUF:]

        def gather_start(g):
            return pltpu.async_copy(
                table_hbm.at[idx_v.at[pl.ds(g * CH, CH)]],
                rows_v.at[g % NBUF],
                gsem[g % NBUF],
            )

        def write_start(g):
            return pltpu.async_copy(
                rows_v.at[g % NBUF],
                out_hbm.at[pl.ds(base + g * CH, CH)],
                wsem[g % NBUF],
            )

        g_h = {}
        w_h = {}
        for g in range(min(NBUF, nchunk)):
            g_h[g] = gather_start(g)
        for g in range(nchunk):
            nxt = g + NBUF - 1
            if NBUF <= nxt < nchunk:
                w_h[nxt - NBUF].wait()
                g_h[nxt] = gather_start(nxt)
            g_h[g].wait()
            w_h[g] = write_start(g)
        for g in range(max(0, nchunk - NBUF), nchunk):
            w_h[g].wait()

    return k


def kernel(x, W_E):
    s, V, D = W_E.shape
    b, c = x.shape
    B = b * c
    table = W_E.reshape(V, D)
    out = _make_gather(V, D, B)(table, x.astype(jnp.int32))
    return out.reshape(b, s, c, D)


# speedup vs baseline: 1.6031x; 1.0000x over previous
import functools

import jax
import jax.numpy as jnp
from jax import lax
from jax.experimental import pallas as pl
from jax.experimental.pallas import tpu as pltpu
from jax.experimental.pallas import tpu_sc as plsc


def _make_gather(V, D, B):
    info = plsc.get_sparse_core_info()
    NC, NS = info.num_cores, info.num_subcores
    NW = NC * NS
    assert B % NW == 0
    b_per_w = B // NW
    CH = 16
    NBUF = 7
    nchunk = b_per_w // CH
    assert nchunk >= NBUF

    mesh = plsc.VectorSubcoreMesh(core_axis_name="c", subcore_axis_name="s")

    @functools.partial(
        pl.kernel,
        mesh=mesh,
        out_type=jax.ShapeDtypeStruct((B, D), jnp.float32),
        scratch_types=[
            pltpu.VMEM((b_per_w,), jnp.int32),
            pltpu.VMEM((NBUF, CH, D), jnp.float32),
        ] + [pltpu.SemaphoreType.DMA] * (2 * NBUF),
    )
    def k(table_hbm, idx_hbm, out_hbm, idx_v, rows_v, *sems):
        wid = lax.axis_index("s") * NC + lax.axis_index("c")
        base = wid * b_per_w
        per_row = idx_hbm.shape[1] // b_per_w
        pltpu.sync_copy(
            idx_hbm.at[wid // per_row, pl.ds((wid % per_row) * b_per_w, b_per_w)],
            idx_v,
        )

        gsem = sems[:NBUF]
        wsem = sems[NBUF:]

        def gather_start(g):
            return pltpu.async_copy(
                table_hbm.at[idx_v.at[pl.ds(g * CH, CH)]],
                rows_v.at[g % NBUF],
                gsem[g % NBUF],
            )

        def write_start(g):
            return pltpu.async_copy(
                rows_v.at[g % NBUF],
                out_hbm.at[pl.ds(base + g * CH, CH)],
                wsem[g % NBUF],
            )

        g_h = {}
        w_h = {}
        for g in range(min(NBUF, nchunk)):
            g_h[g] = gather_start(g)
        for g in range(nchunk):
            nxt = g + NBUF - 1
            if NBUF <= nxt < nchunk:
                w_h[nxt - NBUF].wait()
                g_h[nxt] = gather_start(nxt)
            g_h[g].wait()
            w_h[g] = write_start(g)
        for g in range(max(0, nchunk - NBUF), nchunk):
            w_h[g].wait()

    return k


def kernel(x, W_E):
    s, V, D = W_E.shape
    b, c = x.shape
    B = b * c
    table = W_E.reshape(V, D)
    out = _make_gather(V, D, B)(table, x.astype(jnp.int32))
    return out.reshape(b, s, c, D)
